# Initial kernel scaffold; baseline (speedup 1.0000x reference)
import jax
import jax.numpy as jnp
from jax.experimental import pallas as pl


def kernel(x, pos):
    raise NotImplementedError("write your pallas kernel here")


# speedup vs baseline: 1.8860x; 1.8860x over previous
import jax
import jax.numpy as jnp
from jax.experimental import pallas as pl


def _add_body(x_ref, pos_ref, o_ref):
    o_ref[...] = x_ref[...] + pos_ref[...]


def kernel(x, pos):
    B, T, D = x.shape
    TBLK = 1024
    nt = T // TBLK
    return pl.pallas_call(
        _add_body,
        grid=(nt, B),
        in_specs=[
            pl.BlockSpec((1, TBLK, D), lambda t, b: (b, t, 0)),
            pl.BlockSpec((TBLK, D), lambda t, b: (t, 0)),
        ],
        out_specs=pl.BlockSpec((1, TBLK, D), lambda t, b: (b, t, 0)),
        out_shape=jax.ShapeDtypeStruct(x.shape, x.dtype),
    )(x, pos)


# speedup vs baseline: 1.9914x; 1.0559x over previous
import jax
import jax.numpy as jnp
from jax.experimental import pallas as pl


def _add_body(x_ref, pos_ref, o_ref):
    o_ref[...] = x_ref[...] + pos_ref[...]


def kernel(x, pos):
    B, T, D = x.shape
    TBLK = 2048
    nt = T // TBLK
    return pl.pallas_call(
        _add_body,
        grid=(nt, B),
        in_specs=[
            pl.BlockSpec((1, TBLK, D), lambda t, b: (b, t, 0)),
            pl.BlockSpec((TBLK, D), lambda t, b: (t, 0)),
        ],
        out_specs=pl.BlockSpec((1, TBLK, D), lambda t, b: (b, t, 0)),
        out_shape=jax.ShapeDtypeStruct(x.shape, x.dtype),
    )(x, pos)
